# baseline (speedup 1.0000x reference)
import jax
import jax.numpy as jnp
from jax import lax
from jax.experimental import pallas as pl
from jax.experimental.pallas import tpu as pltpu
from jax.experimental.pallas import tpu_sc as plsc

_NW = 32
_WB = 8
_NBUF = 2
_MESH = plsc.VectorSubcoreMesh(core_axis_name="c", subcore_axis_name="s")


def _body(idx_hbm, pe_hbm, out_hbm, idx_v, rows_v, gsem, wsem):
    sid = lax.axis_index("s")
    wid = sid * 2 + lax.axis_index("c")
    b_per_w = idx_v.shape[0]
    n_groups = b_per_w // (_WB * _NBUF)
    pltpu.sync_copy(idx_hbm.at[wid], idx_v)
    out_base = wid * b_per_w

    def group(t, carry):
        for u in range(_NBUF):
            g = t * _NBUF + u

            @pl.when(t > 0)
            def _(u=u):
                pltpu.make_async_copy(
                    rows_v.at[u], out_hbm.at[pl.ds(0, _WB)],
                    wsem.at[u]).wait()

            gd = [pltpu.async_copy(
                      pe_hbm.at[idx_v.at[g * _WB + k]],
                      rows_v.at[u].at[k], gsem.at[u])
                  for k in range(_WB)]
            for d in gd:
                d.wait()
            pltpu.async_copy(
                rows_v.at[u],
                out_hbm.at[pl.ds(out_base + g * _WB, _WB)],
                wsem.at[u])
        return carry

    lax.fori_loop(0, n_groups, group, 0)
    for u in range(_NBUF):
        pltpu.make_async_copy(
            rows_v.at[u], out_hbm.at[pl.ds(0, _WB)], wsem.at[u]).wait()


def kernel(indices, pe):
    b, l = indices.shape
    d = pe.shape[1]
    b_per_w = b // _NW
    idx = indices.reshape(_NW, b_per_w, l)
    return pl.kernel(
        _body,
        out_type=jax.ShapeDtypeStruct((b, l, d), jnp.float32),
        mesh=_MESH,
        scratch_types=[
            pltpu.VMEM((b_per_w, l), jnp.int32),
            pltpu.VMEM((_NBUF, _WB, l, d), jnp.float32),
            pltpu.SemaphoreType.DMA((_NBUF,)),
            pltpu.SemaphoreType.DMA((_NBUF,)),
        ],
    )(idx, pe)


# speedup vs baseline: 1.0494x; 1.0494x over previous
import jax
import jax.numpy as jnp
from jax import lax
from jax.experimental import pallas as pl
from jax.experimental.pallas import tpu as pltpu
from jax.experimental.pallas import tpu_sc as plsc

_NW = 32
_LP = 56
_GATHER = 112
_WCHUNK = 224
_NBUF = 2
_PERIOD = 2048
_MESH = plsc.VectorSubcoreMesh(core_axis_name="c", subcore_axis_name="s")


def _body(idx_hbm, pe_hbm, out_hbm, idx_v, rows_v, pe_sh, gsem, wsem):
    sid = lax.axis_index("s")
    wid = sid * 2 + lax.axis_index("c")
    n_gathers = idx_v.shape[0]
    gathers_per_chunk = _WCHUNK // _GATHER
    n_chunks = n_gathers // gathers_per_chunk
    n_groups = n_chunks // _NBUF
    rows_per_w = n_gathers * _GATHER
    pltpu.sync_copy(idx_hbm.at[wid], idx_v)
    out_base = wid * rows_per_w

    rows_per_sub = pe_sh.shape[0] // 16
    pltpu.sync_copy(pe_hbm.at[pl.ds(sid * rows_per_sub, rows_per_sub)],
                    pe_sh.at[pl.ds(sid * rows_per_sub, rows_per_sub)])
    plsc.subcore_barrier()

    def group(t, carry):
        for u in range(_NBUF):
            c = t * _NBUF + u

            @pl.when(t > 0)
            def _(u=u):
                pltpu.make_async_copy(
                    rows_v.at[u], out_hbm.at[pl.ds(0, _WCHUNK)],
                    wsem.at[u]).wait()

            gd = [pltpu.async_copy(
                      pe_sh.at[idx_v.at[c * gathers_per_chunk + k]],
                      rows_v.at[u].at[pl.ds(k * _GATHER, _GATHER)],
                      gsem.at[u])
                  for k in range(gathers_per_chunk)]
            for d in gd:
                d.wait()
            pltpu.async_copy(
                rows_v.at[u],
                out_hbm.at[pl.ds(out_base + c * _WCHUNK, _WCHUNK)],
                wsem.at[u])
        return carry

    lax.fori_loop(0, n_groups, group, 0)
    for u in range(_NBUF):
        pltpu.make_async_copy(
            rows_v.at[u], out_hbm.at[pl.ds(0, _WCHUNK)], wsem.at[u]).wait()


def kernel(indices, pe):
    b, l = indices.shape
    d = pe.shape[1]
    idxp = jnp.pad(indices & (_PERIOD - 1), ((0, 0), (0, _LP - l)))
    n_gathers = b * _LP // (_NW * _GATHER)
    idxp = idxp.reshape(_NW, n_gathers, _GATHER)
    out_flat = pl.kernel(
        _body,
        out_type=jax.ShapeDtypeStruct((b * _LP, d), jnp.float32),
        mesh=_MESH,
        scratch_types=[
            pltpu.VMEM((n_gathers, _GATHER), jnp.int32),
            pltpu.VMEM((_NBUF, _WCHUNK, d), jnp.float32),
            pltpu.VMEM_SHARED((_PERIOD, d), jnp.float32),
            pltpu.SemaphoreType.DMA((_NBUF,)),
            pltpu.SemaphoreType.DMA((_NBUF,)),
        ],
    )(idxp, pe)
    return out_flat.reshape(b, _LP, d)[:, :l, :]


# speedup vs baseline: 1.2250x; 1.1673x over previous
import jax
import jax.numpy as jnp
from jax import lax
from jax.experimental import pallas as pl
from jax.experimental.pallas import tpu as pltpu
from jax.experimental.pallas import tpu_sc as plsc

_NW = 32
_LP = 56
_GATHER = 112
_WB = 4
_NBUF = 2
_PERIOD = 2048
_MESH = plsc.VectorSubcoreMesh(core_axis_name="c", subcore_axis_name="s")


def _body(idx_hbm, pe_hbm, out_hbm, idx_v, rows_v, pe_sh, gsem, wsem):
    sid = lax.axis_index("s")
    wid = sid * 2 + lax.axis_index("c")
    l = out_hbm.shape[1]
    n_gathers = idx_v.shape[0]
    gathers_per_chunk = _WB * _LP // _GATHER
    n_chunks = n_gathers // gathers_per_chunk
    n_groups = n_chunks // _NBUF
    b_base = wid * (n_gathers * _GATHER // _LP)
    pltpu.sync_copy(idx_hbm.at[wid], idx_v)

    rows_per_sub = pe_sh.shape[0] // 16
    pltpu.sync_copy(pe_hbm.at[pl.ds(sid * rows_per_sub, rows_per_sub)],
                    pe_sh.at[pl.ds(sid * rows_per_sub, rows_per_sub)])
    plsc.subcore_barrier()

    def wsrc(u):
        return rows_v.at[u].reshape(_WB, _LP, rows_v.shape[-1]).at[
            :, pl.ds(0, l)]

    def group(t, carry):
        for u in range(_NBUF):
            c = t * _NBUF + u

            @pl.when(t > 0)
            def _(u=u):
                pltpu.make_async_copy(
                    wsrc(u), out_hbm.at[pl.ds(0, _WB)], wsem.at[u]).wait()

            gd = [pltpu.async_copy(
                      pe_sh.at[idx_v.at[c * gathers_per_chunk + k]],
                      rows_v.at[u].at[pl.ds(k * _GATHER, _GATHER)],
                      gsem.at[u])
                  for k in range(gathers_per_chunk)]
            for d in gd:
                d.wait()
            pltpu.async_copy(
                wsrc(u), out_hbm.at[pl.ds(b_base + c * _WB, _WB)],
                wsem.at[u])
        return carry

    lax.fori_loop(0, n_groups, group, 0)
    for u in range(_NBUF):
        pltpu.make_async_copy(
            wsrc(u), out_hbm.at[pl.ds(0, _WB)], wsem.at[u]).wait()


def kernel(indices, pe):
    b, l = indices.shape
    d = pe.shape[1]
    idxp = jnp.pad(indices & (_PERIOD - 1), ((0, 0), (0, _LP - l)))
    n_gathers = b * _LP // (_NW * _GATHER)
    idxp = idxp.reshape(_NW, n_gathers, _GATHER)
    return pl.kernel(
        _body,
        out_type=jax.ShapeDtypeStruct((b, l, d), jnp.float32),
        mesh=_MESH,
        scratch_types=[
            pltpu.VMEM((n_gathers, _GATHER), jnp.int32),
            pltpu.VMEM((_NBUF, _WB * _LP, d), jnp.float32),
            pltpu.VMEM_SHARED((_PERIOD, d), jnp.float32),
            pltpu.SemaphoreType.DMA((_NBUF,)),
            pltpu.SemaphoreType.DMA((_NBUF,)),
        ],
    )(idxp, pe)


# speedup vs baseline: 1.2263x; 1.0011x over previous
import jax
import jax.numpy as jnp
from jax import lax
from jax.experimental import pallas as pl
from jax.experimental.pallas import tpu as pltpu
from jax.experimental.pallas import tpu_sc as plsc

_NW = 32
_LP = 56
_GATHER = 112
_WB = 4
_NBUF = 2
_PERIOD = 2048
_MESH = plsc.VectorSubcoreMesh(core_axis_name="c", subcore_axis_name="s")


def _body(idx_hbm, pe_hbm, out_hbm, idx_v, rows_v, pe_sh, gsem, wsem):
    sid = lax.axis_index("s")
    wid = sid * 2 + lax.axis_index("c")
    l = out_hbm.shape[1]
    n_gathers = idx_v.shape[0]
    gathers_per_chunk = _WB * _LP // _GATHER
    n_chunks = n_gathers // gathers_per_chunk
    n_groups = n_chunks // _NBUF
    b_base = wid * (n_gathers * _GATHER // _LP)
    pltpu.sync_copy(idx_hbm.at[wid], idx_v)

    rows_per_sub = pe_sh.shape[0] // 16
    pltpu.sync_copy(pe_hbm.at[pl.ds(sid * rows_per_sub, rows_per_sub)],
                    pe_sh.at[pl.ds(sid * rows_per_sub, rows_per_sub)])
    plsc.subcore_barrier()

    def wsrc(u):
        return rows_v.at[u].reshape(_WB, _LP, rows_v.shape[-1]).at[
            :, pl.ds(0, l)]

    def group(t, carry):
        for u in range(_NBUF):
            c = t * _NBUF + u

            @pl.when(t > 0)
            def _(u=u):
                pltpu.make_async_copy(
                    wsrc(u), out_hbm.at[pl.ds(0, _WB)], wsem.at[u]).wait()

            gd = [pltpu.async_copy(
                      pe_sh.at[idx_v.at[c * gathers_per_chunk + k]],
                      rows_v.at[u].at[pl.ds(k * _GATHER, _GATHER)],
                      gsem.at[u])
                  for k in range(gathers_per_chunk)]
            for d in gd:
                d.wait()
            pltpu.async_copy(
                wsrc(u), out_hbm.at[pl.ds(b_base + c * _WB, _WB)],
                wsem.at[u])
        return carry

    lax.fori_loop(0, n_groups, group, 0)
    for u in range(_NBUF):
        pltpu.make_async_copy(
            wsrc(u), out_hbm.at[pl.ds(0, _WB)], wsem.at[u]).wait()


def kernel(indices, pe):
    b, l = indices.shape
    d = pe.shape[1]
    idxp = jnp.pad(indices & (_PERIOD - 1), ((0, 0), (0, _LP - l)))
    n_gathers = b * _LP // (_NW * _GATHER)
    idxp = idxp.reshape(_NW, n_gathers, _GATHER)
    return pl.kernel(
        _body,
        out_type=jax.ShapeDtypeStruct((b, l, d), jnp.float32),
        mesh=_MESH,
        compiler_params=pltpu.CompilerParams(use_tc_tiling_on_sc=True),
        scratch_types=[
            pltpu.VMEM((n_gathers, _GATHER), jnp.int32),
            pltpu.VMEM((_NBUF, _WB * _LP, d), jnp.float32),
            pltpu.VMEM_SHARED((_PERIOD, d), jnp.float32),
            pltpu.SemaphoreType.DMA((_NBUF,)),
            pltpu.SemaphoreType.DMA((_NBUF,)),
        ],
    )(idxp, pe)
